# baseline (speedup 1.0000x reference)
import jax
import jax.numpy as jnp
from jax.experimental import pallas as pl
from jax.experimental.pallas import tpu as pltpu

_B, _N, _DIN, _DHID, _DOUT = 32, 2048, 32, 16, 64
_EPS = 1e-5
_BLK = 256
_NBLK = _N // _BLK
_HB = _DHID * _B

_f32 = jnp.float32
_bf16 = jnp.bfloat16


def _body_support1(xt_ref, w1b_ref, xw_ref):
    xw_ref[...] = jnp.dot(xt_ref[...], w1b_ref[...],
                          preferred_element_type=_f32).astype(_bf16)


def _body_hop1(adj_ref, xw_ref, w2b_ref, scale_ref, shift_ref, bias1_ref,
               s2_ref):
    adj = adj_ref[...].astype(_bf16)
    h1 = jnp.dot(adj, xw_ref[...],
                 preferred_element_type=_f32) + bias1_ref[...]
    mean = jnp.mean(h1, axis=1, keepdims=True)
    var = jnp.mean(h1 * h1, axis=1, keepdims=True) - mean * mean
    sc = scale_ref[...] * jax.lax.rsqrt(var + _EPS)
    t = shift_ref[...] - mean * sc
    bnh1 = h1 * sc + t
    s2_ref[...] = jnp.dot(bnh1.astype(_bf16), w2b_ref[...],
                          preferred_element_type=_f32).astype(_bf16)


def _body_hop2(adj_ref, s2_ref, fcwp_ref, bias2_ref, fcb_ref, out_ref,
               acc_ref):
    i = pl.program_id(0)
    adj = adj_ref[...].astype(_bf16)
    h2 = jnp.dot(adj, s2_ref[...],
                 preferred_element_type=_f32) + bias2_ref[...]
    part = jnp.zeros((_DOUT, _B), dtype=_f32)
    for h in range(_DHID):
        blk = h2[:, _B * h:_B * (h + 1)].astype(_bf16)
        part = part + jnp.dot(fcwp_ref[h], blk,
                              preferred_element_type=_f32)
    prev = jnp.where(i == 0, jnp.zeros_like(part), acc_ref[...])
    acc = prev + part
    acc_ref[...] = acc
    out_ref[...] = acc.T + fcb_ref[...]


def kernel(x, network, W1, b1, W2, b2, gamma, beta, fcW, fcb):
    xt = jnp.transpose(x, (1, 0, 2)).reshape(_N, _B * _DIN).astype(_bf16)
    eye = jnp.eye(_B, dtype=_f32)
    w1b = (eye[:, None, None, :] * W1[None, :, :, None]).reshape(
        _B * _DIN, _HB).astype(_bf16)
    w2b = jnp.kron(W2, eye).astype(_bf16)
    bias1 = jnp.repeat(b1, _B)[None, :]
    bias2 = jnp.repeat(b2, _B)[None, :]
    fcwp = fcW.reshape(_DOUT, _N, _DHID).transpose(2, 0, 1).astype(_bf16)

    xw = pl.pallas_call(
        _body_support1,
        out_shape=jax.ShapeDtypeStruct((_N, _HB), _bf16),
    )(xt, w1b)

    s2 = pl.pallas_call(
        _body_hop1,
        grid=(_NBLK,),
        in_specs=[
            pl.BlockSpec((_BLK, _N), lambda i: (i, 0)),
            pl.BlockSpec((_N, _HB), lambda i: (0, 0)),
            pl.BlockSpec((_HB, _HB), lambda i: (0, 0)),
            pl.BlockSpec((_BLK, 1), lambda i: (i, 0)),
            pl.BlockSpec((_BLK, 1), lambda i: (i, 0)),
            pl.BlockSpec((1, _HB), lambda i: (0, 0)),
        ],
        out_specs=pl.BlockSpec((_BLK, _HB), lambda i: (i, 0)),
        out_shape=jax.ShapeDtypeStruct((_N, _HB), _bf16),
        compiler_params=pltpu.CompilerParams(
            dimension_semantics=("arbitrary",)),
    )(network, xw, w2b, gamma[:, None], beta[:, None], bias1)

    return pl.pallas_call(
        _body_hop2,
        grid=(_NBLK,),
        in_specs=[
            pl.BlockSpec((_BLK, _N), lambda i: (i, 0)),
            pl.BlockSpec((_N, _HB), lambda i: (0, 0)),
            pl.BlockSpec((_DHID, _DOUT, _BLK),
                         lambda i: (0, 0, i)),
            pl.BlockSpec((1, _HB), lambda i: (0, 0)),
            pl.BlockSpec((1, _DOUT), lambda i: (0, 0)),
        ],
        out_specs=pl.BlockSpec((_B, _DOUT), lambda i: (0, 0)),
        out_shape=jax.ShapeDtypeStruct((_B, _DOUT), _f32),
        scratch_shapes=[pltpu.VMEM((_DOUT, _B), _f32)],
        compiler_params=pltpu.CompilerParams(
            dimension_semantics=("arbitrary",)),
    )(network, s2, fcwp, bias2, fcb[None, :])


# speedup vs baseline: 1.2368x; 1.2368x over previous
import jax
import jax.numpy as jnp
from jax.experimental import pallas as pl
from jax.experimental.pallas import tpu as pltpu

_B, _N, _DIN, _DHID, _DOUT = 32, 2048, 32, 16, 64
_EPS = 1e-5
_BLK = 256
_NBLK = _N // _BLK
_HB = _DHID * _B

_f32 = jnp.float32
_bf16 = jnp.bfloat16


def _iota_eq(shape, fa, fb):
    a = fa(jax.lax.broadcasted_iota(jnp.int32, shape, 0))
    b = fb(jax.lax.broadcasted_iota(jnp.int32, shape, 1))
    return (a == b).astype(_f32)


def _body_prep(xt_ref, w1_ref, w2_ref, b1_ref, b2_ref,
               xw_ref, w2b_ref, bias1_ref, bias2_ref):
    hexp = _iota_eq((_DHID, _HB), lambda r: r, lambda c: c // _B)
    p1 = _iota_eq((_B * _DIN, _DIN), lambda r: r % _DIN, lambda c: c)
    v1 = jnp.dot(p1.astype(_bf16),
                 jnp.dot(w1_ref[...].astype(_bf16), hexp.astype(_bf16),
                         preferred_element_type=_f32).astype(_bf16),
                 preferred_element_type=_f32)
    d1 = _iota_eq((_B * _DIN, _HB), lambda r: r // _DIN, lambda c: c % _B)
    xw_ref[...] = jnp.dot(xt_ref[...], (v1 * d1).astype(_bf16),
                          preferred_element_type=_f32).astype(_bf16)
    p2 = _iota_eq((_HB, _DHID), lambda r: r // _B, lambda c: c)
    v2 = jnp.dot(p2.astype(_bf16),
                 jnp.dot(w2_ref[...].astype(_bf16), hexp.astype(_bf16),
                         preferred_element_type=_f32).astype(_bf16),
                 preferred_element_type=_f32)
    d2 = _iota_eq((_HB, _HB), lambda r: r % _B, lambda c: c % _B)
    w2b_ref[...] = (v2 * d2).astype(_bf16)
    bias1_ref[...] = jnp.dot(b1_ref[...], hexp, preferred_element_type=_f32)
    bias2_ref[...] = jnp.dot(b2_ref[...], hexp, preferred_element_type=_f32)


def _body_main(adjr_ref, adjc_ref, xw_ref, w2b_ref, scale_ref, shift_ref,
               bias1_ref, bias2_ref, fcwp_ref, fcb_ref, out_ref, h2_ref):
    i = pl.program_id(0)

    @pl.when(i < _NBLK)
    def _sweep():
        h1 = jnp.dot(adjr_ref[...].astype(_bf16), xw_ref[...],
                     preferred_element_type=_f32) + bias1_ref[...]
        mean = jnp.mean(h1, axis=1, keepdims=True)
        var = jnp.mean(h1 * h1, axis=1, keepdims=True) - mean * mean
        sc = scale_ref[...] * jax.lax.rsqrt(var + _EPS)
        t = shift_ref[...] - mean * sc
        bnh1 = h1 * sc + t
        s2j = jnp.dot(bnh1.astype(_bf16), w2b_ref[...],
                      preferred_element_type=_f32).astype(_bf16)
        part = jnp.dot(adjc_ref[...].astype(_bf16), s2j,
                       preferred_element_type=_f32)
        prev = jnp.where(i == 0, jnp.zeros_like(part), h2_ref[...])
        h2_ref[...] = prev + part

    @pl.when(i == _NBLK)
    def _readout():
        h2 = h2_ref[...] + bias2_ref[...]
        acc = jnp.zeros((_DOUT, _B), dtype=_f32)
        for h in range(_DHID):
            blk = h2[:, _B * h:_B * (h + 1)].astype(_bf16)
            acc = acc + jnp.dot(fcwp_ref[h], blk,
                                preferred_element_type=_f32)
        out_ref[...] = acc.T + fcb_ref[...]


def kernel(x, network, W1, b1, W2, b2, gamma, beta, fcW, fcb):
    xt = jnp.transpose(x, (1, 0, 2)).reshape(_N, _B * _DIN).astype(_bf16)
    fcwp = fcW.reshape(_DOUT, _N, _DHID).transpose(2, 0, 1).astype(_bf16)

    xw, w2b, bias1, bias2 = pl.pallas_call(
        _body_prep,
        out_shape=(
            jax.ShapeDtypeStruct((_N, _HB), _bf16),
            jax.ShapeDtypeStruct((_HB, _HB), _bf16),
            jax.ShapeDtypeStruct((1, _HB), _f32),
            jax.ShapeDtypeStruct((1, _HB), _f32),
        ),
    )(xt, W1, W2, b1[None, :], b2[None, :])

    clamp = _NBLK - 1
    return pl.pallas_call(
        _body_main,
        grid=(_NBLK + 1,),
        in_specs=[
            pl.BlockSpec((_BLK, _N), lambda i: (jnp.minimum(i, clamp), 0)),
            pl.BlockSpec((_N, _BLK), lambda i: (0, jnp.minimum(i, clamp))),
            pl.BlockSpec((_N, _HB), lambda i: (0, 0)),
            pl.BlockSpec((_HB, _HB), lambda i: (0, 0)),
            pl.BlockSpec((_BLK, 1), lambda i: (jnp.minimum(i, clamp), 0)),
            pl.BlockSpec((_BLK, 1), lambda i: (jnp.minimum(i, clamp), 0)),
            pl.BlockSpec((1, _HB), lambda i: (0, 0)),
            pl.BlockSpec((1, _HB), lambda i: (0, 0)),
            pl.BlockSpec((_DHID, _DOUT, _N), lambda i: (0, 0, 0)),
            pl.BlockSpec((1, _DOUT), lambda i: (0, 0)),
        ],
        out_specs=pl.BlockSpec((_B, _DOUT), lambda i: (0, 0)),
        out_shape=jax.ShapeDtypeStruct((_B, _DOUT), _f32),
        scratch_shapes=[pltpu.VMEM((_N, _HB), _f32)],
        compiler_params=pltpu.CompilerParams(
            dimension_semantics=("arbitrary",)),
    )(network, network, xw, w2b, gamma[:, None], beta[:, None],
      bias1, bias2, fcwp, fcb[None, :])
